# baseline (speedup 1.0000x reference)
import functools
import math

import jax
import jax.numpy as jnp
import numpy as np
from jax.experimental import pallas as pl

_B, _S, _D, _H = 1, 2048, 1024, 16
_QL, _KVL, _NOPE, _ROPE, _VH = 1536, 512, 128, 64, 128
_QH = _NOPE + _ROPE
_INTER = 512
_E, _NSH, _NG, _TKG, _TOPK = 16, 2, 4, 2, 4
_GS = _E // _NG


def _ln_body(x, w, b, eps=1e-5):
    m = jnp.mean(x, -1, keepdims=True)
    v = jnp.mean((x - m) ** 2, -1, keepdims=True)
    return (x - m) / jnp.sqrt(v + eps) * w + b


def _dot_t(a, bt):
    return jax.lax.dot_general(a, bt, (((1,), (1,)), ((), ())),
                               preferred_element_type=jnp.float32)


def _ln_mm_kernel(x_ref, lnw_ref, lnb_ref, wt_ref, o_ref):
    h = _ln_body(x_ref[...], lnw_ref[0], lnb_ref[0])
    o_ref[...] = _dot_t(h, wt_ref[...])


def _ln_mm(x, lnw, lnb, wt, bn=512):
    m, k = x.shape
    n = wt.shape[0]
    grid = (pl.cdiv(n, bn),)
    return pl.pallas_call(
        _ln_mm_kernel,
        grid=grid,
        in_specs=[
            pl.BlockSpec((m, k), lambda i: (0, 0)),
            pl.BlockSpec((1, k), lambda i: (0, 0)),
            pl.BlockSpec((1, k), lambda i: (0, 0)),
            pl.BlockSpec((bn, k), lambda i: (i, 0)),
        ],
        out_specs=pl.BlockSpec((m, bn), lambda i: (0, i)),
        out_shape=jax.ShapeDtypeStruct((m, n), jnp.float32),
    )(x, lnw.reshape(1, k), lnb.reshape(1, k), wt)


def _attn_kernel(qn_ref, qp_ref, qpr_ref, kn_ref, kp_ref, kpr_ref, v_ref,
                 cos_ref, sin_ref, o_ref):
    cos = cos_ref[...]
    sin = sin_ref[...]
    qpe = qp_ref[0] * cos + qpr_ref[0] * sin
    kpe = kp_ref[...] * cos + kpr_ref[...] * sin
    s = _dot_t(qn_ref[0], kn_ref[0]) + _dot_t(qpe, kpe)
    s = s * (1.0 / math.sqrt(_QH))
    row = jax.lax.broadcasted_iota(jnp.int32, (_S, _S), 0)
    col = jax.lax.broadcasted_iota(jnp.int32, (_S, _S), 1)
    s = jnp.where(col > row, jnp.float32(-1e9), s)
    m = jnp.max(s, axis=1, keepdims=True)
    p = jnp.exp(s - m)
    p = p / jnp.sum(p, axis=1, keepdims=True)
    o_ref[0] = jax.lax.dot_general(p, v_ref[0], (((1,), (0,)), ((), ())),
                                   preferred_element_type=jnp.float32)


def _attention(qn, qp, qpr, kn, kp, kpr, v, cos2, sin2):
    return pl.pallas_call(
        _attn_kernel,
        grid=(_H,),
        in_specs=[
            pl.BlockSpec((1, _S, _NOPE), lambda h: (h, 0, 0)),
            pl.BlockSpec((1, _S, _ROPE), lambda h: (h, 0, 0)),
            pl.BlockSpec((1, _S, _ROPE), lambda h: (h, 0, 0)),
            pl.BlockSpec((1, _S, _NOPE), lambda h: (h, 0, 0)),
            pl.BlockSpec((_S, _ROPE), lambda h: (0, 0)),
            pl.BlockSpec((_S, _ROPE), lambda h: (0, 0)),
            pl.BlockSpec((1, _S, _VH), lambda h: (h, 0, 0)),
            pl.BlockSpec((_S, _ROPE), lambda h: (0, 0)),
            pl.BlockSpec((_S, _ROPE), lambda h: (0, 0)),
        ],
        out_specs=pl.BlockSpec((1, _S, _VH), lambda h: (h, 0, 0)),
        out_shape=jax.ShapeDtypeStruct((_H, _S, _VH), jnp.float32),
    )(qn, qp, qpr, kn, kp, kpr, v, cos2, sin2)


def _oproj_kernel(ao_ref, ow_ref, x_ref, ln2w_ref, ln2b_ref, h1_ref, xf_ref):
    h1 = x_ref[...] + _dot_t(ao_ref[...], ow_ref[...])
    h1_ref[...] = h1
    xf_ref[...] = _ln_body(h1, ln2w_ref[0], ln2b_ref[0])


def _oproj(ao, o_w, x, ln2w, ln2b, bm=1024):
    grid = (_S // bm,)
    return pl.pallas_call(
        _oproj_kernel,
        grid=grid,
        in_specs=[
            pl.BlockSpec((bm, _H * _VH), lambda i: (i, 0)),
            pl.BlockSpec((_D, _H * _VH), lambda i: (0, 0)),
            pl.BlockSpec((bm, _D), lambda i: (i, 0)),
            pl.BlockSpec((1, _D), lambda i: (0, 0)),
            pl.BlockSpec((1, _D), lambda i: (0, 0)),
        ],
        out_specs=[
            pl.BlockSpec((bm, _D), lambda i: (i, 0)),
            pl.BlockSpec((bm, _D), lambda i: (i, 0)),
        ],
        out_shape=[
            jax.ShapeDtypeStruct((_S, _D), jnp.float32),
            jax.ShapeDtypeStruct((_S, _D), jnp.float32),
        ],
    )(ao, o_w, x, ln2w.reshape(1, _D), ln2b.reshape(1, _D))


def _route_kernel(xf_ref, gw_ref, comb_ref):
    bm = xf_ref.shape[0]
    l = _dot_t(xf_ref[...], gw_ref[...])
    ivec = jax.lax.broadcasted_iota(jnp.int32, (bm, _E), 1)
    r = jnp.zeros((bm, _E), jnp.float32)
    for j in range(_E):
        lj = jax.lax.slice_in_dim(l, j, j + 1, axis=1)
        beats = (lj > l) | ((lj == l) & (j < ivec))
        sg = (ivec // _GS) == (j // _GS)
        r = r + jnp.where(beats & sg, 1.0, 0.0)
    cand = r < _TKG
    pos = (ivec // _GS).astype(jnp.float32) * _TKG + r
    rr = jnp.zeros((bm, _E), jnp.float32)
    for j in range(_E):
        lj = jax.lax.slice_in_dim(l, j, j + 1, axis=1)
        pj = jax.lax.slice_in_dim(pos, j, j + 1, axis=1)
        cj = jax.lax.slice_in_dim(cand, j, j + 1, axis=1)
        beats2 = cj & ((lj > l) | ((lj == l) & (pj < pos)))
        rr = rr + jnp.where(beats2, 1.0, 0.0)
    sel = cand & (rr < _TOPK)
    w = jnp.where(sel, l, jnp.float32(0.0))
    comb_ref[...] = w / (jnp.sum(w, axis=1, keepdims=True) + 1e-20)


def _route(xf, gate_w, bm=512):
    return pl.pallas_call(
        _route_kernel,
        grid=(_S // bm,),
        in_specs=[
            pl.BlockSpec((bm, _D), lambda i: (i, 0)),
            pl.BlockSpec((_E, _D), lambda i: (0, 0)),
        ],
        out_specs=pl.BlockSpec((bm, _E), lambda i: (i, 0)),
        out_shape=jax.ShapeDtypeStruct((_S, _E), jnp.float32),
    )(xf, gate_w)


def _moe_kernel(xf_ref, h1_ref, comb_ref, gu_ref, dn_ref, o_ref):
    e = pl.program_id(0)

    @pl.when(e == 0)
    def _():
        o_ref[...] = h1_ref[...]

    h = _dot_t(xf_ref[...], gu_ref[0])
    g = h[:, :_INTER]
    u = h[:, _INTER:]
    act = (g / (1.0 + jnp.exp(-g))) * u
    y = _dot_t(act, dn_ref[0])
    ne = comb_ref.shape[1]
    lane = jax.lax.broadcasted_iota(jnp.int32, (xf_ref.shape[0], ne), 1)
    c = jnp.sum(jnp.where(lane == e, comb_ref[...], 0.0), axis=1, keepdims=True)
    o_ref[...] += c * y


def _moe(xf, h1, comb, gu_all, dn_all):
    ne = gu_all.shape[0]
    return pl.pallas_call(
        _moe_kernel,
        grid=(ne,),
        in_specs=[
            pl.BlockSpec((_S, _D), lambda e: (0, 0)),
            pl.BlockSpec((_S, _D), lambda e: (0, 0)),
            pl.BlockSpec((_S, ne), lambda e: (0, 0)),
            pl.BlockSpec((1, 2 * _INTER, _D), lambda e: (e, 0, 0)),
            pl.BlockSpec((1, _D, _INTER), lambda e: (e, 0, 0)),
        ],
        out_specs=pl.BlockSpec((_S, _D), lambda e: (0, 0)),
        out_shape=jax.ShapeDtypeStruct((_S, _D), jnp.float32),
    )(xf, h1, comb, gu_all, dn_all)


def _rope_tables():
    inv = 1.0 / (10000.0 ** (np.arange(0, _ROPE, 2, dtype=np.float64) / _ROPE))
    t = np.arange(_S, dtype=np.float64)
    f = np.outer(t, inv)
    cos = np.cos(f)
    sin = np.sin(f)
    cos2 = np.repeat(cos, 2, axis=1)
    sin2 = np.repeat(sin, 2, axis=1)
    return jnp.asarray(cos2, jnp.float32), jnp.asarray(sin2, jnp.float32)


def _rot_pairs(x):
    xe = x[..., 0::2]
    xo = x[..., 1::2]
    return jnp.stack([-xo, xe], axis=-1).reshape(x.shape)


def kernel(x, ln1_w, ln1_b, ln2_w, ln2_b, qkv_a_w, qa_ln_w, qa_ln_b, q_b_w,
           kv_a_w, kv_ln_w, kv_ln_b, kv_b_w, o_w, gate_w, exp_gu, exp_dn,
           sh_gu, sh_dn):
    x2 = x.reshape(_S, _D)
    cos2, sin2 = _rope_tables()

    w1 = jnp.concatenate([qkv_a_w[:_QL], kv_a_w], axis=0)
    out1 = _ln_mm(x2, ln1_w, ln1_b, w1)
    q_lat = out1[:, :_QL]
    c_kv = out1[:, _QL:_QL + _KVL]
    k_pe = out1[:, _QL + _KVL:]

    q = _ln_mm(q_lat, qa_ln_w, qa_ln_b, q_b_w)
    kvb = _ln_mm(c_kv, kv_ln_w, kv_ln_b, kv_b_w)

    q3 = q.reshape(_S, _H, _QH).transpose(1, 0, 2)
    qn = q3[:, :, :_NOPE]
    qp = q3[:, :, _NOPE:]
    qpr = _rot_pairs(qp)
    kvb3 = kvb.reshape(_S, _H, _NOPE + _VH).transpose(1, 0, 2)
    kn = kvb3[:, :, :_NOPE]
    v3 = kvb3[:, :, _NOPE:]
    kpr = _rot_pairs(k_pe)

    ao = _attention(qn, qp, qpr, kn, k_pe, kpr, v3, cos2, sin2)
    ao2 = ao.transpose(1, 0, 2).reshape(_S, _H * _VH)

    h1, xf = _oproj(ao2, o_w, x2, ln2_w, ln2_b)

    comb = _route(xf, gate_w)

    comb_all = jnp.concatenate(
        [jnp.ones((_S, _NSH), jnp.float32), comb], axis=1)
    gu_all = jnp.concatenate([sh_gu, exp_gu], axis=0)
    dn_all = jnp.concatenate([sh_dn, exp_dn], axis=0)
    out = _moe(xf, h1, comb_all, gu_all, dn_all)

    return out.reshape(_B, _S, _D)
